# Initial kernel scaffold; baseline (speedup 1.0000x reference)
import jax
import jax.numpy as jnp
from jax.experimental import pallas as pl


def kernel(node_feats, edge_index, edge_attr, edge_type, node_type_ids, edge_type_ids, W_node_types, W_edge_attr, W_edge_type, W_node_update, W_att):
    raise NotImplementedError("write your pallas kernel here")


# speedup vs baseline: 1.8853x; 1.8853x over previous
import functools

import jax
import jax.numpy as jnp
from jax.experimental import pallas as pl

N = 1024
E = 32768
NODE_EMB = 64
OUT = 8
HEADS = 4
N_NODE_TYPES = 3

A2_BLK = 1024
C_BLK = 8


def _a1_body(nf_ref, tid_ref, wnt_ref, wsrc_ref, wtar_ref, tsrc_ref, ttar_ref):
    nf = nf_ref[...]
    tid = tid_ref[...]
    ne = jnp.zeros((N, NODE_EMB), dtype=jnp.float32)
    for t in range(N_NODE_TYPES):
        emb_t = jnp.dot(nf, wnt_ref[t], preferred_element_type=jnp.float32)
        ne = jnp.where(tid == t, emb_t, ne)
    tsrc_ref[...] = jnp.dot(ne, wsrc_ref[...], preferred_element_type=jnp.float32)
    ttar_ref[...] = jnp.dot(ne, wtar_ref[...], preferred_element_type=jnp.float32)


def _leaky(x):
    return jnp.where(x >= 0, x, 0.2 * x)


def _a2_body(src_ref, dst_ref, ea_ref, et_ref, wea_ref, wet_ref,
             waea_ref, waet_ref, wuea_ref, tsrc_ref, ttar_ref, rec_ref):
    ea_emb = _leaky(jnp.dot(ea_ref[...], wea_ref[...],
                            preferred_element_type=jnp.float32))
    et_emb = _leaky(jnp.dot(et_ref[...], wet_ref[...],
                            preferred_element_type=jnp.float32))
    es = (jnp.dot(ea_emb, waea_ref[...], preferred_element_type=jnp.float32)
          + jnp.dot(et_emb, waet_ref[...], preferred_element_type=jnp.float32))
    mu = jnp.dot(ea_emb, wuea_ref[...], preferred_element_type=jnp.float32)

    src = src_ref[...].reshape(A2_BLK, 1)
    dst = dst_ref[...].reshape(A2_BLK, 1)
    node_iota = jax.lax.broadcasted_iota(jnp.int32, (A2_BLK, N), 1)
    oh_src = (node_iota == src).astype(jnp.float32)
    g_src = jnp.dot(oh_src, tsrc_ref[...], preferred_element_type=jnp.float32)
    oh_dst = (node_iota == dst).astype(jnp.float32)
    g_tar = jnp.dot(oh_dst, ttar_ref[...], preferred_element_type=jnp.float32)

    score = _leaky(es + g_src[:, 0:4] + g_tar[:, 0:4])
    msg = _leaky(mu + g_src[:, 4:12])
    rec_ref[...] = jnp.concatenate(
        [score, msg, jnp.zeros((A2_BLK, 4), jnp.float32)], axis=1)


def _c_body(s_ref, m_ref, o_ref):
    s = s_ref[...]
    sm = jnp.where(s == 0.0, -10000.0, s)
    mx = jnp.max(sm, axis=1, keepdims=True)
    e = jnp.exp(sm - mx)
    z = jnp.sum(e, axis=1)
    msg = m_ref[...]
    u = jax.lax.dot_general(e, msg, (((1,), (1,)), ((0,), (0,))),
                            preferred_element_type=jnp.float32)
    o_ref[...] = u / z[:, :, None]


def kernel(node_feats, edge_index, edge_attr, edge_type, node_type_ids,
           edge_type_ids, W_node_types, W_edge_attr, W_edge_type,
           W_node_update, W_att):
    f32 = jnp.float32
    wa_tar = W_att[0:NODE_EMB]
    wa_ea = W_att[NODE_EMB:NODE_EMB + 32]
    wa_et = W_att[NODE_EMB + 32:NODE_EMB + 64]
    wa_src = W_att[NODE_EMB + 64:]
    wu_src = W_node_update[:NODE_EMB]
    wu_ea = W_node_update[NODE_EMB:]
    zeros64_4 = jnp.zeros((NODE_EMB, 4), f32)
    wsrc16 = jnp.concatenate([wa_src, wu_src, zeros64_4], axis=1)
    wtar16 = jnp.concatenate([wa_tar, jnp.zeros((NODE_EMB, 12), f32)], axis=1)

    tsrc, ttar = pl.pallas_call(
        _a1_body,
        out_shape=(jax.ShapeDtypeStruct((N, 16), f32),
                   jax.ShapeDtypeStruct((N, 16), f32)),
    )(node_feats, node_type_ids.reshape(N, 1), W_node_types, wsrc16, wtar16)

    n_blk = E // A2_BLK
    src3 = edge_index[0].reshape(n_blk, A2_BLK, 1)
    dst3 = edge_index[1].reshape(n_blk, A2_BLK, 1)
    records = pl.pallas_call(
        _a2_body,
        grid=(n_blk,),
        in_specs=[
            pl.BlockSpec((1, A2_BLK, 1), lambda i: (i, 0, 0)),
            pl.BlockSpec((1, A2_BLK, 1), lambda i: (i, 0, 0)),
            pl.BlockSpec((A2_BLK, 16), lambda i: (i, 0)),
            pl.BlockSpec((A2_BLK, 8), lambda i: (i, 0)),
            pl.BlockSpec((16, 32), lambda i: (0, 0)),
            pl.BlockSpec((8, 32), lambda i: (0, 0)),
            pl.BlockSpec((32, 4), lambda i: (0, 0)),
            pl.BlockSpec((32, 4), lambda i: (0, 0)),
            pl.BlockSpec((32, 8), lambda i: (0, 0)),
            pl.BlockSpec((N, 16), lambda i: (0, 0)),
            pl.BlockSpec((N, 16), lambda i: (0, 0)),
        ],
        out_specs=pl.BlockSpec((A2_BLK, 16), lambda i: (i, 0)),
        out_shape=jax.ShapeDtypeStruct((E, 16), f32),
    )(src3, dst3, edge_attr, edge_type, W_edge_attr, W_edge_type,
      wa_ea, wa_et, wu_ea, tsrc, ttar)

    src, dst = edge_index[0], edge_index[1]
    smat = jnp.zeros((N, N, HEADS), f32).at[src, dst].add(records[:, 0:4])
    mmat = jnp.zeros((N, N, OUT), f32).at[src, dst].add(records[:, 4:12])

    out = pl.pallas_call(
        _c_body,
        grid=(N // C_BLK,),
        in_specs=[
            pl.BlockSpec((C_BLK, N, HEADS), lambda i: (i, 0, 0)),
            pl.BlockSpec((C_BLK, N, OUT), lambda i: (i, 0, 0)),
        ],
        out_specs=pl.BlockSpec((C_BLK, HEADS, OUT), lambda i: (i, 0, 0)),
        out_shape=jax.ShapeDtypeStruct((N, HEADS, OUT), f32),
    )(smat, mmat)
    return out.reshape(N, HEADS * OUT)


# speedup vs baseline: 2.0195x; 1.0711x over previous
import functools

import jax
import jax.numpy as jnp
from jax import lax
from jax.experimental import pallas as pl
from jax.experimental.pallas import tpu as pltpu
from jax.experimental.pallas import tpu_sc as plsc

N = 1024
E = 32768
NODE_EMB = 64
OUT = 8
HEADS = 4
N_NODE_TYPES = 3

A2_BLK = 1024
C_BLK = 8

SC_SUBCORES = 16
SC_CORES = 2
EDGES_PER_TILE = E // SC_SUBCORES
BLK_ROWS = 32
N_BLOCKS = N // BLK_ROWS
BLK_SLOTS = BLK_ROWS * N
ROWS_PER_TILE = BLK_SLOTS // SC_SUBCORES


def _a1_body(nf_ref, tid_ref, wnt_ref, wsrc_ref, wtar_ref, tsrc_ref, ttar_ref):
    nf = nf_ref[...]
    tid = tid_ref[...]
    ne = jnp.zeros((N, NODE_EMB), dtype=jnp.float32)
    for t in range(N_NODE_TYPES):
        emb_t = jnp.dot(nf, wnt_ref[t], preferred_element_type=jnp.float32)
        ne = jnp.where(tid == t, emb_t, ne)
    tsrc_ref[...] = jnp.dot(ne, wsrc_ref[...], preferred_element_type=jnp.float32)
    ttar_ref[...] = jnp.dot(ne, wtar_ref[...], preferred_element_type=jnp.float32)


def _leaky(x):
    return jnp.where(x >= 0, x, 0.2 * x)


def _a2_body(src_ref, dst_ref, ea_ref, et_ref, wea_ref, wet_ref,
             waea_ref, waet_ref, wuea_ref, tsrc_ref, ttar_ref, rec_ref):
    ea_emb = _leaky(jnp.dot(ea_ref[...], wea_ref[...],
                            preferred_element_type=jnp.float32))
    et_emb = _leaky(jnp.dot(et_ref[...], wet_ref[...],
                            preferred_element_type=jnp.float32))
    es = (jnp.dot(ea_emb, waea_ref[...], preferred_element_type=jnp.float32)
          + jnp.dot(et_emb, waet_ref[...], preferred_element_type=jnp.float32))
    mu = jnp.dot(ea_emb, wuea_ref[...], preferred_element_type=jnp.float32)

    src = src_ref[...].reshape(A2_BLK, 1)
    dst = dst_ref[...].reshape(A2_BLK, 1)
    node_iota = jax.lax.broadcasted_iota(jnp.int32, (A2_BLK, N), 1)
    oh_src = (node_iota == src).astype(jnp.float32)
    g_src = jnp.dot(oh_src, tsrc_ref[...], preferred_element_type=jnp.float32)
    oh_dst = (node_iota == dst).astype(jnp.float32)
    g_tar = jnp.dot(oh_dst, ttar_ref[...], preferred_element_type=jnp.float32)

    score = _leaky(es + g_src[:, 0:4] + g_tar[:, 0:4])
    msg = _leaky(mu + g_src[:, 4:12])
    rec_ref[...] = jnp.concatenate(
        [score, msg, jnp.zeros((A2_BLK, 4), jnp.float32)], axis=1)


def _b_body(ei_ref, rec_hbm, zeros_hbm, out_ref,
            rec_v, zeros_v, src_v, dst_v, idx_v, shared):
    cid = lax.axis_index("c")
    sid = lax.axis_index("s")
    e0 = sid * EDGES_PER_TILE
    pltpu.sync_copy(rec_hbm.at[pl.ds(e0, EDGES_PER_TILE)], rec_v)
    pltpu.sync_copy(ei_ref.at[0, pl.ds(e0, EDGES_PER_TILE)], src_v)
    pltpu.sync_copy(ei_ref.at[1, pl.ds(e0, EDGES_PER_TILE)], dst_v)
    pltpu.sync_copy(zeros_hbm, zeros_v)
    junk = BLK_SLOTS + sid

    def per_block(b, carry):
        gb = cid * (N_BLOCKS // SC_CORES) + b
        pltpu.sync_copy(zeros_v, shared.at[pl.ds(sid * ROWS_PER_TILE, ROWS_PER_TILE)])
        plsc.subcore_barrier()

        def per_chunk(c, carry2):
            s16 = src_v[pl.ds(c * 16, 16)]
            d16 = dst_v[pl.ds(c * 16, 16)]
            inb = (s16 >> 5) == gb
            local = ((s16 & (BLK_ROWS - 1)) << 10) | d16
            iv = jnp.where(inb, local, junk)
            idx_v[c >> 3, pl.ds((c & 7) * 16, 16)] = iv
            return carry2
        lax.fori_loop(0, EDGES_PER_TILE // 16, per_chunk, 0)

        for j in range(EDGES_PER_TILE // 128):
            pltpu.sync_copy(rec_v.at[pl.ds(j * 128, 128)],
                            shared.at[idx_v.at[j]], add=True)
        plsc.subcore_barrier()
        pltpu.sync_copy(
            shared.at[pl.ds(sid * ROWS_PER_TILE, ROWS_PER_TILE)],
            out_ref.at[pl.ds(gb * BLK_SLOTS + sid * ROWS_PER_TILE, ROWS_PER_TILE)])
        plsc.subcore_barrier()
        return carry

    lax.fori_loop(0, N_BLOCKS // SC_CORES, per_block, 0)


def _c_body(d_ref, o_ref):
    x = d_ref[...].reshape(C_BLK, N, 16)
    s = x[:, :, 0:4]
    sm = jnp.where(s == 0.0, -10000.0, s)
    mx = jnp.max(sm, axis=1, keepdims=True)
    e = jnp.exp(sm - mx)
    z = jnp.sum(e, axis=1)
    msg = x[:, :, 4:12]
    u = jax.lax.dot_general(e, msg, (((1,), (1,)), ((0,), (0,))),
                            preferred_element_type=jnp.float32)
    o_ref[...] = u / z[:, :, None]


def kernel(node_feats, edge_index, edge_attr, edge_type, node_type_ids,
           edge_type_ids, W_node_types, W_edge_attr, W_edge_type,
           W_node_update, W_att):
    f32 = jnp.float32
    wa_tar = W_att[0:NODE_EMB]
    wa_ea = W_att[NODE_EMB:NODE_EMB + 32]
    wa_et = W_att[NODE_EMB + 32:NODE_EMB + 64]
    wa_src = W_att[NODE_EMB + 64:]
    wu_src = W_node_update[:NODE_EMB]
    wu_ea = W_node_update[NODE_EMB:]
    zeros64_4 = jnp.zeros((NODE_EMB, 4), f32)
    wsrc16 = jnp.concatenate([wa_src, wu_src, zeros64_4], axis=1)
    wtar16 = jnp.concatenate([wa_tar, jnp.zeros((NODE_EMB, 12), f32)], axis=1)

    tsrc, ttar = pl.pallas_call(
        _a1_body,
        out_shape=(jax.ShapeDtypeStruct((N, 16), f32),
                   jax.ShapeDtypeStruct((N, 16), f32)),
    )(node_feats, node_type_ids.reshape(N, 1), W_node_types, wsrc16, wtar16)

    n_blk = E // A2_BLK
    src3 = edge_index[0].reshape(n_blk, A2_BLK, 1)
    dst3 = edge_index[1].reshape(n_blk, A2_BLK, 1)
    records = pl.pallas_call(
        _a2_body,
        grid=(n_blk,),
        in_specs=[
            pl.BlockSpec((1, A2_BLK, 1), lambda i: (i, 0, 0)),
            pl.BlockSpec((1, A2_BLK, 1), lambda i: (i, 0, 0)),
            pl.BlockSpec((A2_BLK, 16), lambda i: (i, 0)),
            pl.BlockSpec((A2_BLK, 8), lambda i: (i, 0)),
            pl.BlockSpec((16, 32), lambda i: (0, 0)),
            pl.BlockSpec((8, 32), lambda i: (0, 0)),
            pl.BlockSpec((32, 4), lambda i: (0, 0)),
            pl.BlockSpec((32, 4), lambda i: (0, 0)),
            pl.BlockSpec((32, 8), lambda i: (0, 0)),
            pl.BlockSpec((N, 16), lambda i: (0, 0)),
            pl.BlockSpec((N, 16), lambda i: (0, 0)),
        ],
        out_specs=pl.BlockSpec((A2_BLK, 16), lambda i: (i, 0)),
        out_shape=jax.ShapeDtypeStruct((E, 16), f32),
    )(src3, dst3, edge_attr, edge_type, W_edge_attr, W_edge_type,
      wa_ea, wa_et, wu_ea, tsrc, ttar)

    zeros_blk = jnp.zeros((ROWS_PER_TILE, 16), f32)
    mesh = plsc.VectorSubcoreMesh(core_axis_name="c", subcore_axis_name="s")
    dense = pl.kernel(
        _b_body,
        out_type=jax.ShapeDtypeStruct((N * N, 16), f32),
        mesh=mesh,
        compiler_params=pltpu.CompilerParams(use_tc_tiling_on_sc=False),
        scratch_types=[
            pltpu.VMEM((EDGES_PER_TILE, 16), f32),
            pltpu.VMEM((ROWS_PER_TILE, 16), f32),
            pltpu.VMEM((EDGES_PER_TILE,), jnp.int32),
            pltpu.VMEM((EDGES_PER_TILE,), jnp.int32),
            pltpu.VMEM((EDGES_PER_TILE // 128, 128), jnp.int32),
            pltpu.VMEM_SHARED((BLK_SLOTS + SC_SUBCORES, 16), f32),
        ],
    )(edge_index, records, zeros_blk)

    out = pl.pallas_call(
        _c_body,
        grid=(N // C_BLK,),
        in_specs=[pl.BlockSpec((C_BLK * N, 16), lambda i: (i, 0))],
        out_specs=pl.BlockSpec((C_BLK, HEADS, OUT), lambda i: (i, 0, 0)),
        out_shape=jax.ShapeDtypeStruct((N, HEADS, OUT), f32),
    )(dense)
    return out.reshape(N, HEADS * OUT)


# speedup vs baseline: 2.2578x; 1.1180x over previous
import functools

import jax
import jax.numpy as jnp
from jax import lax
from jax.experimental import pallas as pl
from jax.experimental.pallas import tpu as pltpu
from jax.experimental.pallas import tpu_sc as plsc

N = 1024
E = 32768
NODE_EMB = 64
OUT = 8
HEADS = 4
N_NODE_TYPES = 3

A2_BLK = 1024
C_BLK = 32

SC_SUBCORES = 16
SC_CORES = 2
EDGES_PER_TILE = E // SC_SUBCORES
BLK_ROWS = 32
N_BLOCKS = N // BLK_ROWS
BLK_SLOTS = BLK_ROWS * N
ROWS_PER_TILE = BLK_SLOTS // SC_SUBCORES


def _a1_body(nf_ref, tid_ref, wnt_ref, wsrc_ref, wtar_ref, tsrc_ref, ttar_ref):
    nf = nf_ref[...]
    tid = tid_ref[...]
    ne = jnp.zeros((N, NODE_EMB), dtype=jnp.float32)
    for t in range(N_NODE_TYPES):
        emb_t = jnp.dot(nf, wnt_ref[t], preferred_element_type=jnp.float32)
        ne = jnp.where(tid == t, emb_t, ne)
    tsrc_ref[...] = jnp.dot(ne, wsrc_ref[...], preferred_element_type=jnp.float32)
    ttar_ref[...] = jnp.dot(ne, wtar_ref[...], preferred_element_type=jnp.float32)


def _leaky(x):
    return jnp.where(x >= 0, x, 0.2 * x)


def _a2_body(src_ref, dst_ref, ea_ref, et_ref, wea_ref, wet_ref,
             waea_ref, waet_ref, wuea_ref, tsrc_ref, ttar_ref, rec_ref):
    ea_emb = _leaky(jnp.dot(ea_ref[...], wea_ref[...],
                            preferred_element_type=jnp.float32))
    et_emb = _leaky(jnp.dot(et_ref[...], wet_ref[...],
                            preferred_element_type=jnp.float32))
    es = (jnp.dot(ea_emb, waea_ref[...], preferred_element_type=jnp.float32)
          + jnp.dot(et_emb, waet_ref[...], preferred_element_type=jnp.float32))
    mu = jnp.dot(ea_emb, wuea_ref[...], preferred_element_type=jnp.float32)

    src = src_ref[...].reshape(A2_BLK, 1)
    dst = dst_ref[...].reshape(A2_BLK, 1)
    node_iota = jax.lax.broadcasted_iota(jnp.int32, (A2_BLK, N), 1)
    oh_src = (node_iota == src).astype(jnp.float32)
    g_src = jnp.dot(oh_src, tsrc_ref[...], preferred_element_type=jnp.float32)
    oh_dst = (node_iota == dst).astype(jnp.float32)
    g_tar = jnp.dot(oh_dst, ttar_ref[...], preferred_element_type=jnp.float32)

    score = _leaky(es + g_src[:, 0:4] + g_tar[:, 0:4])
    msg = _leaky(mu + g_src[:, 4:12])
    rec_ref[...] = jnp.concatenate(
        [score, msg, jnp.zeros((A2_BLK, 4), jnp.float32)], axis=1)


def _b_body(ei_ref, rec_hbm, zeros_hbm, out_ref,
            rec_v, zeros_v, src_v, dst_v, idx_v, shared):
    cid = lax.axis_index("c")
    sid = lax.axis_index("s")
    e0 = sid * EDGES_PER_TILE
    pltpu.sync_copy(rec_hbm.at[pl.ds(e0, EDGES_PER_TILE)], rec_v)
    pltpu.sync_copy(ei_ref.at[0, pl.ds(e0, EDGES_PER_TILE)], src_v)
    pltpu.sync_copy(ei_ref.at[1, pl.ds(e0, EDGES_PER_TILE)], dst_v)
    pltpu.sync_copy(zeros_hbm, zeros_v)
    junk = BLK_SLOTS + sid

    def per_block(b, carry):
        gb = cid * (N_BLOCKS // SC_CORES) + b
        pltpu.sync_copy(zeros_v, shared.at[pl.ds(sid * ROWS_PER_TILE, ROWS_PER_TILE)])
        plsc.subcore_barrier()

        def per_chunk(c, carry2):
            s16 = src_v[pl.ds(c * 16, 16)]
            d16 = dst_v[pl.ds(c * 16, 16)]
            inb = (s16 >> 5) == gb
            local = ((s16 & (BLK_ROWS - 1)) << 10) | d16
            iv = jnp.where(inb, local, junk)
            idx_v[c >> 3, pl.ds((c & 7) * 16, 16)] = iv
            return carry2
        lax.fori_loop(0, EDGES_PER_TILE // 16, per_chunk, 0)

        for j in range(EDGES_PER_TILE // 128):
            pltpu.sync_copy(rec_v.at[pl.ds(j * 128, 128)],
                            shared.at[idx_v.at[j]], add=True)
        plsc.subcore_barrier()
        pltpu.sync_copy(
            shared.at[pl.ds(sid * ROWS_PER_TILE, ROWS_PER_TILE)],
            out_ref.at[pl.ds(gb * BLK_SLOTS + sid * ROWS_PER_TILE, ROWS_PER_TILE)])
        plsc.subcore_barrier()
        return carry

    lax.fori_loop(0, N_BLOCKS // SC_CORES, per_block, 0)


def _c_body(d_ref, o_ref):
    x = d_ref[...]
    xt = x.T
    s = jnp.stack([xt[h].reshape(C_BLK, N) for h in range(HEADS)],
                  axis=0)
    sm = jnp.where(s == 0.0, -10000.0, s)
    mx = jnp.max(sm, axis=2, keepdims=True)
    e = jnp.exp(sm - mx)
    z = jnp.sum(e, axis=2)
    msg = jnp.stack([xt[HEADS + o].reshape(C_BLK, N) for o in range(OUT)],
                    axis=0)
    ucols = []
    for h in range(HEADS):
        for o in range(OUT):
            ucols.append(jnp.sum(e[h] * msg[o], axis=1))
    u = jnp.stack(ucols, axis=1).reshape(C_BLK, HEADS, OUT)
    o_ref[...] = u / z.T[:, :, None]


def kernel(node_feats, edge_index, edge_attr, edge_type, node_type_ids,
           edge_type_ids, W_node_types, W_edge_attr, W_edge_type,
           W_node_update, W_att):
    f32 = jnp.float32
    wa_tar = W_att[0:NODE_EMB]
    wa_ea = W_att[NODE_EMB:NODE_EMB + 32]
    wa_et = W_att[NODE_EMB + 32:NODE_EMB + 64]
    wa_src = W_att[NODE_EMB + 64:]
    wu_src = W_node_update[:NODE_EMB]
    wu_ea = W_node_update[NODE_EMB:]
    zeros64_4 = jnp.zeros((NODE_EMB, 4), f32)
    wsrc16 = jnp.concatenate([wa_src, wu_src, zeros64_4], axis=1)
    wtar16 = jnp.concatenate([wa_tar, jnp.zeros((NODE_EMB, 12), f32)], axis=1)

    tsrc, ttar = pl.pallas_call(
        _a1_body,
        out_shape=(jax.ShapeDtypeStruct((N, 16), f32),
                   jax.ShapeDtypeStruct((N, 16), f32)),
    )(node_feats, node_type_ids.reshape(N, 1), W_node_types, wsrc16, wtar16)

    n_blk = E // A2_BLK
    src3 = edge_index[0].reshape(n_blk, A2_BLK, 1)
    dst3 = edge_index[1].reshape(n_blk, A2_BLK, 1)
    records = pl.pallas_call(
        _a2_body,
        grid=(n_blk,),
        in_specs=[
            pl.BlockSpec((1, A2_BLK, 1), lambda i: (i, 0, 0)),
            pl.BlockSpec((1, A2_BLK, 1), lambda i: (i, 0, 0)),
            pl.BlockSpec((A2_BLK, 16), lambda i: (i, 0)),
            pl.BlockSpec((A2_BLK, 8), lambda i: (i, 0)),
            pl.BlockSpec((16, 32), lambda i: (0, 0)),
            pl.BlockSpec((8, 32), lambda i: (0, 0)),
            pl.BlockSpec((32, 4), lambda i: (0, 0)),
            pl.BlockSpec((32, 4), lambda i: (0, 0)),
            pl.BlockSpec((32, 8), lambda i: (0, 0)),
            pl.BlockSpec((N, 16), lambda i: (0, 0)),
            pl.BlockSpec((N, 16), lambda i: (0, 0)),
        ],
        out_specs=pl.BlockSpec((A2_BLK, 16), lambda i: (i, 0)),
        out_shape=jax.ShapeDtypeStruct((E, 16), f32),
    )(src3, dst3, edge_attr, edge_type, W_edge_attr, W_edge_type,
      wa_ea, wa_et, wu_ea, tsrc, ttar)

    zeros_blk = jnp.zeros((ROWS_PER_TILE, 16), f32)
    mesh = plsc.VectorSubcoreMesh(core_axis_name="c", subcore_axis_name="s")
    dense = pl.kernel(
        _b_body,
        out_type=jax.ShapeDtypeStruct((N * N, 16), f32),
        mesh=mesh,
        compiler_params=pltpu.CompilerParams(use_tc_tiling_on_sc=False),
        scratch_types=[
            pltpu.VMEM((EDGES_PER_TILE, 16), f32),
            pltpu.VMEM((ROWS_PER_TILE, 16), f32),
            pltpu.VMEM((EDGES_PER_TILE,), jnp.int32),
            pltpu.VMEM((EDGES_PER_TILE,), jnp.int32),
            pltpu.VMEM((EDGES_PER_TILE // 128, 128), jnp.int32),
            pltpu.VMEM_SHARED((BLK_SLOTS + SC_SUBCORES, 16), f32),
        ],
    )(edge_index, records, zeros_blk)

    out = pl.pallas_call(
        _c_body,
        grid=(N // C_BLK,),
        in_specs=[pl.BlockSpec((C_BLK * N, 16), lambda i: (i, 0))],
        out_specs=pl.BlockSpec((C_BLK, HEADS, OUT), lambda i: (i, 0, 0)),
        out_shape=jax.ShapeDtypeStruct((N, HEADS, OUT), f32),
    )(dense)
    return out.reshape(N, HEADS * OUT)


# speedup vs baseline: 4.7835x; 2.1186x over previous
import functools

import jax
import jax.numpy as jnp
from jax import lax
from jax.experimental import pallas as pl
from jax.experimental.pallas import tpu as pltpu
from jax.experimental.pallas import tpu_sc as plsc

N = 1024
E = 32768
NODE_EMB = 64
OUT = 8
HEADS = 4
N_NODE_TYPES = 3

A2_BLK = 1024
C_BLK = 32

SC_SUBCORES = 16
SC_CORES = 2
EDGES_PER_TILE = E // SC_SUBCORES
BLK_ROWS = 32
N_BLOCKS = N // BLK_ROWS
BLK_SLOTS = BLK_ROWS * N
ROWS_PER_TILE = BLK_SLOTS // SC_SUBCORES


def _a1_body(nf_ref, tid_ref, wnt_ref, wsrc_ref, wtar_ref, tsrc_ref, ttar_ref):
    nf = nf_ref[...]
    tid = tid_ref[...]
    ne = jnp.zeros((N, NODE_EMB), dtype=jnp.float32)
    for t in range(N_NODE_TYPES):
        emb_t = jnp.dot(nf, wnt_ref[t], preferred_element_type=jnp.float32)
        ne = jnp.where(tid == t, emb_t, ne)
    tsrc_ref[...] = jnp.dot(ne, wsrc_ref[...], preferred_element_type=jnp.float32)
    ttar_ref[...] = jnp.dot(ne, wtar_ref[...], preferred_element_type=jnp.float32)


def _leaky(x):
    return jnp.where(x >= 0, x, 0.2 * x)


def _a2_body(src_ref, dst_ref, ea_ref, et_ref, wea_ref, wet_ref,
             waea_ref, waet_ref, wuea_ref, tsrc_ref, ttar_ref, rec_ref):
    ea_emb = _leaky(jnp.dot(ea_ref[...], wea_ref[...],
                            preferred_element_type=jnp.float32))
    et_emb = _leaky(jnp.dot(et_ref[...], wet_ref[...],
                            preferred_element_type=jnp.float32))
    es = (jnp.dot(ea_emb, waea_ref[...], preferred_element_type=jnp.float32)
          + jnp.dot(et_emb, waet_ref[...], preferred_element_type=jnp.float32))
    mu = jnp.dot(ea_emb, wuea_ref[...], preferred_element_type=jnp.float32)

    src = src_ref[...].reshape(A2_BLK, 1)
    dst = dst_ref[...].reshape(A2_BLK, 1)
    node_iota = jax.lax.broadcasted_iota(jnp.int32, (A2_BLK, N), 1)
    oh_src = (node_iota == src).astype(jnp.float32)
    g_src = jnp.dot(oh_src, tsrc_ref[...], preferred_element_type=jnp.float32)
    oh_dst = (node_iota == dst).astype(jnp.float32)
    g_tar = jnp.dot(oh_dst, ttar_ref[...], preferred_element_type=jnp.float32)

    score = _leaky(es + g_src[:, 0:4] + g_tar[:, 0:4])
    msg = _leaky(mu + g_src[:, 4:12])
    rec_ref[...] = jnp.concatenate(
        [score, msg, jnp.zeros((A2_BLK, 4), jnp.float32)], axis=1)


def _b_body(ei_ref, rec_hbm, zeros_hbm, out_ref,
            rec_v, zeros_v, src_v, dst_v, idx_v, shared):
    cid = lax.axis_index("c")
    sid = lax.axis_index("s")
    e0 = sid * EDGES_PER_TILE
    pltpu.sync_copy(rec_hbm.at[pl.ds(e0, EDGES_PER_TILE)], rec_v)
    pltpu.sync_copy(ei_ref.at[0, pl.ds(e0, EDGES_PER_TILE)], src_v)
    pltpu.sync_copy(ei_ref.at[1, pl.ds(e0, EDGES_PER_TILE)], dst_v)
    pltpu.sync_copy(zeros_hbm, zeros_v)
    junk = BLK_SLOTS + sid

    def per_block(b, carry):
        gb = cid * (N_BLOCKS // SC_CORES) + b
        pltpu.sync_copy(zeros_v, shared.at[pl.ds(sid * ROWS_PER_TILE, ROWS_PER_TILE)])
        plsc.subcore_barrier()

        def per_chunk(c, carry2):
            s16 = src_v[pl.ds(c * 16, 16)]
            d16 = dst_v[pl.ds(c * 16, 16)]
            inb = (s16 >> 5) == gb
            local = ((s16 & (BLK_ROWS - 1)) << 10) | d16
            iv = jnp.where(inb, local, junk)
            idx_v[c >> 3, pl.ds((c & 7) * 16, 16)] = iv
            return carry2
        lax.fori_loop(0, EDGES_PER_TILE // 16, per_chunk, 0)

        for j in range(EDGES_PER_TILE // 128):
            pltpu.sync_copy(rec_v.at[pl.ds(j * 128, 128)],
                            shared.at[idx_v.at[j]], add=True)
        plsc.subcore_barrier()
        pltpu.sync_copy(
            shared.at[pl.ds(sid * ROWS_PER_TILE, ROWS_PER_TILE)],
            out_ref.at[pl.ds(gb * BLK_SLOTS + sid * ROWS_PER_TILE, ROWS_PER_TILE)])
        plsc.subcore_barrier()
        return carry

    lax.fori_loop(0, N_BLOCKS // SC_CORES, per_block, 0)


def _c_body(d_ref, o_ref):
    x = d_ref[...]
    xt = x.T
    y = xt.reshape(8, 16, C_BLK * N // 8)
    lanes = N // 8

    def plane(c):
        return y[:, c, :].reshape(8, C_BLK, lanes)

    s = jnp.stack([plane(h) for h in range(HEADS)], axis=0)
    sm = jnp.where(s == 0.0, -10000.0, s)
    mx = jnp.max(jnp.max(sm, axis=3), axis=1)
    e = jnp.exp(sm - mx[:, None, :, None])
    z = jnp.sum(jnp.sum(e, axis=3), axis=1)
    m = jnp.stack([plane(HEADS + o) for o in range(OUT)], axis=0)
    ucols = []
    for h in range(HEADS):
        for o in range(OUT):
            t = e[h] * m[o]
            ucols.append(jnp.sum(jnp.sum(t, axis=2), axis=0))
    u = jnp.stack(ucols, axis=1).reshape(C_BLK, HEADS, OUT)
    o_ref[...] = u / z.T[:, :, None]


def kernel(node_feats, edge_index, edge_attr, edge_type, node_type_ids,
           edge_type_ids, W_node_types, W_edge_attr, W_edge_type,
           W_node_update, W_att):
    f32 = jnp.float32
    wa_tar = W_att[0:NODE_EMB]
    wa_ea = W_att[NODE_EMB:NODE_EMB + 32]
    wa_et = W_att[NODE_EMB + 32:NODE_EMB + 64]
    wa_src = W_att[NODE_EMB + 64:]
    wu_src = W_node_update[:NODE_EMB]
    wu_ea = W_node_update[NODE_EMB:]
    zeros64_4 = jnp.zeros((NODE_EMB, 4), f32)
    wsrc16 = jnp.concatenate([wa_src, wu_src, zeros64_4], axis=1)
    wtar16 = jnp.concatenate([wa_tar, jnp.zeros((NODE_EMB, 12), f32)], axis=1)

    tsrc, ttar = pl.pallas_call(
        _a1_body,
        out_shape=(jax.ShapeDtypeStruct((N, 16), f32),
                   jax.ShapeDtypeStruct((N, 16), f32)),
    )(node_feats, node_type_ids.reshape(N, 1), W_node_types, wsrc16, wtar16)

    n_blk = E // A2_BLK
    src3 = edge_index[0].reshape(n_blk, A2_BLK, 1)
    dst3 = edge_index[1].reshape(n_blk, A2_BLK, 1)
    records = pl.pallas_call(
        _a2_body,
        grid=(n_blk,),
        in_specs=[
            pl.BlockSpec((1, A2_BLK, 1), lambda i: (i, 0, 0)),
            pl.BlockSpec((1, A2_BLK, 1), lambda i: (i, 0, 0)),
            pl.BlockSpec((A2_BLK, 16), lambda i: (i, 0)),
            pl.BlockSpec((A2_BLK, 8), lambda i: (i, 0)),
            pl.BlockSpec((16, 32), lambda i: (0, 0)),
            pl.BlockSpec((8, 32), lambda i: (0, 0)),
            pl.BlockSpec((32, 4), lambda i: (0, 0)),
            pl.BlockSpec((32, 4), lambda i: (0, 0)),
            pl.BlockSpec((32, 8), lambda i: (0, 0)),
            pl.BlockSpec((N, 16), lambda i: (0, 0)),
            pl.BlockSpec((N, 16), lambda i: (0, 0)),
        ],
        out_specs=pl.BlockSpec((A2_BLK, 16), lambda i: (i, 0)),
        out_shape=jax.ShapeDtypeStruct((E, 16), f32),
    )(src3, dst3, edge_attr, edge_type, W_edge_attr, W_edge_type,
      wa_ea, wa_et, wu_ea, tsrc, ttar)

    zeros_blk = jnp.zeros((ROWS_PER_TILE, 16), f32)
    mesh = plsc.VectorSubcoreMesh(core_axis_name="c", subcore_axis_name="s")
    dense = pl.kernel(
        _b_body,
        out_type=jax.ShapeDtypeStruct((N * N, 16), f32),
        mesh=mesh,
        compiler_params=pltpu.CompilerParams(use_tc_tiling_on_sc=False),
        scratch_types=[
            pltpu.VMEM((EDGES_PER_TILE, 16), f32),
            pltpu.VMEM((ROWS_PER_TILE, 16), f32),
            pltpu.VMEM((EDGES_PER_TILE,), jnp.int32),
            pltpu.VMEM((EDGES_PER_TILE,), jnp.int32),
            pltpu.VMEM((EDGES_PER_TILE // 128, 128), jnp.int32),
            pltpu.VMEM_SHARED((BLK_SLOTS + SC_SUBCORES, 16), f32),
        ],
    )(edge_index, records, zeros_blk)

    out = pl.pallas_call(
        _c_body,
        grid=(N // C_BLK,),
        in_specs=[pl.BlockSpec((C_BLK * N // 8, 128), lambda i: (i, 0))],
        out_specs=pl.BlockSpec((C_BLK, HEADS, OUT), lambda i: (i, 0, 0)),
        out_shape=jax.ShapeDtypeStruct((N, HEADS, OUT), f32),
    )(dense.reshape(N * N // 8, 128))
    return out.reshape(N, HEADS * OUT)


# speedup vs baseline: 5.1061x; 1.0674x over previous
import functools

import jax
import jax.numpy as jnp
from jax import lax
from jax.experimental import pallas as pl
from jax.experimental.pallas import tpu as pltpu
from jax.experimental.pallas import tpu_sc as plsc

N = 1024
E = 32768
NODE_EMB = 64
OUT = 8
HEADS = 4
N_NODE_TYPES = 3

A2_BLK = 1024
C_BLK = 32

SC_SUBCORES = 16
SC_CORES = 2
EDGES_PER_TILE = E // SC_SUBCORES
BLK_ROWS = 32
N_BLOCKS = N // BLK_ROWS
BLK_SLOTS = BLK_ROWS * N
ROWS_PER_TILE = BLK_SLOTS // SC_SUBCORES


def _a1_body(nf_ref, tid_ref, wnt_ref, wall_ref, tab_ref):
    nf = nf_ref[...]
    tid = tid_ref[...]
    ne = jnp.zeros((N, NODE_EMB), dtype=jnp.float32)
    for t in range(N_NODE_TYPES):
        emb_t = jnp.dot(nf, wnt_ref[t], preferred_element_type=jnp.float32)
        ne = jnp.where(tid == t, emb_t, ne)
    tab_ref[...] = jnp.dot(ne, wall_ref[...], preferred_element_type=jnp.float32)


def _leaky(x):
    return jnp.where(x >= 0, x, 0.2 * x)


def _a2_body(ea_ref, et_ref, wea_ref, wet_ref,
             waea_ref, waet_ref, wuea_ref, rec_ref):
    ea_emb = _leaky(jnp.dot(ea_ref[...], wea_ref[...],
                            preferred_element_type=jnp.float32))
    et_emb = _leaky(jnp.dot(et_ref[...], wet_ref[...],
                            preferred_element_type=jnp.float32))
    es = (jnp.dot(ea_emb, waea_ref[...], preferred_element_type=jnp.float32)
          + jnp.dot(et_emb, waet_ref[...], preferred_element_type=jnp.float32))
    mu = jnp.dot(ea_emb, wuea_ref[...], preferred_element_type=jnp.float32)
    rec_ref[...] = jnp.concatenate(
        [es, mu, jnp.zeros((A2_BLK, 4), jnp.float32)], axis=1)


def _b_body(ei_ref, rec_hbm, tab_hbm, out_ref,
            rec_v, zeros_v, src_v, dst_v, idx_v, tab_v, shared):
    cid = lax.axis_index("c")
    sid = lax.axis_index("s")
    e0 = sid * EDGES_PER_TILE
    pltpu.sync_copy(rec_hbm.at[pl.ds(e0, EDGES_PER_TILE)], rec_v)
    pltpu.sync_copy(ei_ref.at[0, pl.ds(e0, EDGES_PER_TILE)], src_v)
    pltpu.sync_copy(ei_ref.at[1, pl.ds(e0, EDGES_PER_TILE)], dst_v)
    pltpu.sync_copy(tab_hbm, tab_v)
    junk = BLK_SLOTS + sid

    zk = jnp.zeros((16,), jnp.float32)

    def zero_row(r, carry0):
        zeros_v[r, pl.ds(0, 16)] = zk
        return carry0
    lax.fori_loop(0, ROWS_PER_TILE, zero_row, 0)

    lane = jax.lax.iota(jnp.int32, 16)

    def finish_chunk(c, carry0):
        rows = lane + c * 16
        s16 = src_v[pl.ds(c * 16, 16)]
        d16 = dst_v[pl.ds(c * 16, 16)]
        for ch in range(12):
            chv = jnp.full((16,), ch, jnp.int32)
            v = plsc.load_gather(rec_v, [rows, chv])
            v = v + plsc.load_gather(tab_v, [s16, chv])
            if ch < HEADS:
                v = v + plsc.load_gather(tab_v, [d16, chv + 12])
            v = jnp.where(v >= 0, v, 0.2 * v)
            plsc.store_scatter(rec_v, [rows, chv], v)
        return carry0
    lax.fori_loop(0, EDGES_PER_TILE // 16, finish_chunk, 0)

    def per_block(b, carry):
        gb = cid * (N_BLOCKS // SC_CORES) + b
        pltpu.sync_copy(zeros_v, shared.at[pl.ds(sid * ROWS_PER_TILE, ROWS_PER_TILE)])
        plsc.subcore_barrier()

        def per_chunk(c, carry2):
            s16 = src_v[pl.ds(c * 16, 16)]
            d16 = dst_v[pl.ds(c * 16, 16)]
            inb = (s16 >> 5) == gb
            local = ((s16 & (BLK_ROWS - 1)) << 10) | d16
            iv = jnp.where(inb, local, junk)
            idx_v[c >> 3, pl.ds((c & 7) * 16, 16)] = iv
            return carry2
        lax.fori_loop(0, EDGES_PER_TILE // 16, per_chunk, 0)

        for j in range(EDGES_PER_TILE // 128):
            pltpu.sync_copy(rec_v.at[pl.ds(j * 128, 128)],
                            shared.at[idx_v.at[j]], add=True)
        plsc.subcore_barrier()
        pltpu.sync_copy(
            shared.at[pl.ds(sid * ROWS_PER_TILE, ROWS_PER_TILE)],
            out_ref.at[pl.ds(gb * BLK_SLOTS + sid * ROWS_PER_TILE, ROWS_PER_TILE)])
        plsc.subcore_barrier()
        return carry

    lax.fori_loop(0, N_BLOCKS // SC_CORES, per_block, 0)


def _c_body(d_ref, o_ref):
    x = d_ref[...]
    xt = x.T
    y = xt.reshape(8, 16, C_BLK * N // 8)
    lanes = N // 8

    def plane(c):
        return y[:, c, :].reshape(8, C_BLK, lanes)

    s = jnp.stack([plane(h) for h in range(HEADS)], axis=0)
    sm = jnp.where(s == 0.0, -10000.0, s)
    mx = jnp.max(jnp.max(sm, axis=3), axis=1)
    e = jnp.exp(sm - mx[:, None, :, None])
    z = jnp.sum(jnp.sum(e, axis=3), axis=1)
    m = jnp.stack([plane(HEADS + o) for o in range(OUT)], axis=0)
    ucols = []
    for h in range(HEADS):
        for o in range(OUT):
            t = e[h] * m[o]
            ucols.append(jnp.sum(jnp.sum(t, axis=2), axis=0))
    u = jnp.stack(ucols, axis=1).reshape(C_BLK, HEADS, OUT)
    o_ref[...] = u / z.T[:, :, None]


def kernel(node_feats, edge_index, edge_attr, edge_type, node_type_ids,
           edge_type_ids, W_node_types, W_edge_attr, W_edge_type,
           W_node_update, W_att):
    f32 = jnp.float32
    wa_tar = W_att[0:NODE_EMB]
    wa_ea = W_att[NODE_EMB:NODE_EMB + 32]
    wa_et = W_att[NODE_EMB + 32:NODE_EMB + 64]
    wa_src = W_att[NODE_EMB + 64:]
    wu_src = W_node_update[:NODE_EMB]
    wu_ea = W_node_update[NODE_EMB:]
    w_all = jnp.concatenate([wa_src, wu_src, wa_tar], axis=1)

    tab = pl.pallas_call(
        _a1_body,
        out_shape=jax.ShapeDtypeStruct((N, 16), f32),
    )(node_feats, node_type_ids.reshape(N, 1), W_node_types, w_all)

    n_blk = E // A2_BLK
    records = pl.pallas_call(
        _a2_body,
        grid=(n_blk,),
        in_specs=[
            pl.BlockSpec((A2_BLK, 16), lambda i: (i, 0)),
            pl.BlockSpec((A2_BLK, 8), lambda i: (i, 0)),
            pl.BlockSpec((16, 32), lambda i: (0, 0)),
            pl.BlockSpec((8, 32), lambda i: (0, 0)),
            pl.BlockSpec((32, 4), lambda i: (0, 0)),
            pl.BlockSpec((32, 4), lambda i: (0, 0)),
            pl.BlockSpec((32, 8), lambda i: (0, 0)),
        ],
        out_specs=pl.BlockSpec((A2_BLK, 16), lambda i: (i, 0)),
        out_shape=jax.ShapeDtypeStruct((E, 16), f32),
    )(edge_attr, edge_type, W_edge_attr, W_edge_type, wa_ea, wa_et, wu_ea)

    mesh = plsc.VectorSubcoreMesh(core_axis_name="c", subcore_axis_name="s")
    dense = pl.kernel(
        _b_body,
        out_type=jax.ShapeDtypeStruct((N * N, 16), f32),
        mesh=mesh,
        compiler_params=pltpu.CompilerParams(use_tc_tiling_on_sc=False,
                                             needs_layout_passes=False),
        scratch_types=[
            pltpu.VMEM((EDGES_PER_TILE, 16), f32),
            pltpu.VMEM((ROWS_PER_TILE, 16), f32),
            pltpu.VMEM((EDGES_PER_TILE,), jnp.int32),
            pltpu.VMEM((EDGES_PER_TILE,), jnp.int32),
            pltpu.VMEM((EDGES_PER_TILE // 128, 128), jnp.int32),
            pltpu.VMEM((N, 16), f32),
            pltpu.VMEM_SHARED((BLK_SLOTS + SC_SUBCORES, 16), f32),
        ],
    )(edge_index, records, tab)

    out = pl.pallas_call(
        _c_body,
        grid=(N // C_BLK,),
        in_specs=[pl.BlockSpec((C_BLK * N // 8, 128), lambda i: (i, 0))],
        out_specs=pl.BlockSpec((C_BLK, HEADS, OUT), lambda i: (i, 0, 0)),
        out_shape=jax.ShapeDtypeStruct((N, HEADS, OUT), f32),
    )(dense.reshape(N * N // 8, 128))
    return out.reshape(N, HEADS * OUT)


# speedup vs baseline: 5.1563x; 1.0098x over previous
import functools

import jax
import jax.numpy as jnp
from jax import lax
from jax.experimental import pallas as pl
from jax.experimental.pallas import tpu as pltpu
from jax.experimental.pallas import tpu_sc as plsc

N = 1024
E = 32768
NODE_EMB = 64
OUT = 8
HEADS = 4
N_NODE_TYPES = 3

A2_BLK = 1024
C_BLK = 64

SC_SUBCORES = 16
SC_CORES = 2
EDGES_PER_TILE = E // SC_SUBCORES
BLK_ROWS = 32
N_BLOCKS = N // BLK_ROWS
BLK_SLOTS = BLK_ROWS * N
ROWS_PER_TILE = BLK_SLOTS // SC_SUBCORES


def _a1_body(nf_ref, tid_ref, wnt_ref, wall_ref, tab_ref):
    nf = nf_ref[...]
    tid = tid_ref[...]
    ne = jnp.zeros((N, NODE_EMB), dtype=jnp.float32)
    for t in range(N_NODE_TYPES):
        emb_t = jnp.dot(nf, wnt_ref[t], preferred_element_type=jnp.float32)
        ne = jnp.where(tid == t, emb_t, ne)
    tab_ref[...] = jnp.dot(ne, wall_ref[...], preferred_element_type=jnp.float32)


def _leaky(x):
    return jnp.where(x >= 0, x, 0.2 * x)


def _a2_body(ea_ref, et_ref, wea_ref, wet_ref,
             waea_ref, waet_ref, wuea_ref, rec_ref):
    ea_emb = _leaky(jnp.dot(ea_ref[...], wea_ref[...],
                            preferred_element_type=jnp.float32))
    et_emb = _leaky(jnp.dot(et_ref[...], wet_ref[...],
                            preferred_element_type=jnp.float32))
    es = (jnp.dot(ea_emb, waea_ref[...], preferred_element_type=jnp.float32)
          + jnp.dot(et_emb, waet_ref[...], preferred_element_type=jnp.float32))
    mu = jnp.dot(ea_emb, wuea_ref[...], preferred_element_type=jnp.float32)
    rec_ref[...] = jnp.concatenate(
        [es, mu, jnp.zeros((A2_BLK, 4), jnp.float32)], axis=1)


def _b_body(ei_ref, rec_hbm, tab_hbm, out_ref,
            rec_v, zeros_v, src_v, dst_v, idx_v, tab_v, shared):
    cid = lax.axis_index("c")
    sid = lax.axis_index("s")
    e0 = sid * EDGES_PER_TILE
    pltpu.sync_copy(rec_hbm.at[pl.ds(e0, EDGES_PER_TILE)], rec_v)
    pltpu.sync_copy(ei_ref.at[0, pl.ds(e0, EDGES_PER_TILE)], src_v)
    pltpu.sync_copy(ei_ref.at[1, pl.ds(e0, EDGES_PER_TILE)], dst_v)
    pltpu.sync_copy(tab_hbm, tab_v)
    junk = BLK_SLOTS + sid

    zk = jnp.zeros((16,), jnp.float32)

    def zero_row(r, carry0):
        zeros_v[r, pl.ds(0, 16)] = zk
        return carry0
    lax.fori_loop(0, ROWS_PER_TILE, zero_row, 0)

    lane = jax.lax.iota(jnp.int32, 16)

    def finish_chunk(c, carry0):
        rows = lane + c * 16
        s16 = src_v[pl.ds(c * 16, 16)]
        d16 = dst_v[pl.ds(c * 16, 16)]
        for ch in range(12):
            chv = jnp.full((16,), ch, jnp.int32)
            v = plsc.load_gather(rec_v, [rows, chv])
            v = v + plsc.load_gather(tab_v, [s16, chv])
            if ch < HEADS:
                v = v + plsc.load_gather(tab_v, [d16, chv + 12])
            v = jnp.where(v >= 0, v, 0.2 * v)
            plsc.store_scatter(rec_v, [rows, chv], v)
        return carry0
    lax.fori_loop(0, EDGES_PER_TILE // 16, finish_chunk, 0)

    def per_block(b, carry):
        gb = cid * (N_BLOCKS // SC_CORES) + b
        pltpu.sync_copy(zeros_v, shared.at[pl.ds(sid * ROWS_PER_TILE, ROWS_PER_TILE)])
        plsc.subcore_barrier()

        def per_chunk(c, carry2):
            s16 = src_v[pl.ds(c * 16, 16)]
            d16 = dst_v[pl.ds(c * 16, 16)]
            inb = (s16 >> 5) == gb
            local = ((s16 & (BLK_ROWS - 1)) << 10) | d16
            iv = jnp.where(inb, local, junk)
            idx_v[c >> 3, pl.ds((c & 7) * 16, 16)] = iv
            return carry2
        lax.fori_loop(0, EDGES_PER_TILE // 16, per_chunk, 0)

        for j in range(EDGES_PER_TILE // 128):
            pltpu.sync_copy(rec_v.at[pl.ds(j * 128, 128)],
                            shared.at[idx_v.at[j]], add=True)
        plsc.subcore_barrier()
        pltpu.sync_copy(
            shared.at[pl.ds(sid * ROWS_PER_TILE, ROWS_PER_TILE)],
            out_ref.at[pl.ds(gb * BLK_SLOTS + sid * ROWS_PER_TILE, ROWS_PER_TILE)])
        plsc.subcore_barrier()
        return carry

    lax.fori_loop(0, N_BLOCKS // SC_CORES, per_block, 0)


def _c_body(d_ref, o_ref):
    x = d_ref[...]
    xt = x.T
    y = xt.reshape(8, 16, C_BLK * N // 8)
    lanes = N // 8

    def plane(c):
        return y[:, c, :].reshape(8, C_BLK, lanes)

    s = jnp.stack([plane(h) for h in range(HEADS)], axis=0)
    sm = jnp.where(s == 0.0, -10000.0, s)
    mx = jnp.max(jnp.max(sm, axis=3), axis=1)
    e = jnp.exp(sm - mx[:, None, :, None])
    z = jnp.sum(jnp.sum(e, axis=3), axis=1)
    m = jnp.stack([plane(HEADS + o) for o in range(OUT)], axis=0)
    ucols = []
    for h in range(HEADS):
        for o in range(OUT):
            t = e[h] * m[o]
            ucols.append(jnp.sum(jnp.sum(t, axis=2), axis=0))
    u = jnp.stack(ucols, axis=1).reshape(C_BLK, HEADS, OUT)
    o_ref[...] = u / z.T[:, :, None]


def kernel(node_feats, edge_index, edge_attr, edge_type, node_type_ids,
           edge_type_ids, W_node_types, W_edge_attr, W_edge_type,
           W_node_update, W_att):
    f32 = jnp.float32
    wa_tar = W_att[0:NODE_EMB]
    wa_ea = W_att[NODE_EMB:NODE_EMB + 32]
    wa_et = W_att[NODE_EMB + 32:NODE_EMB + 64]
    wa_src = W_att[NODE_EMB + 64:]
    wu_src = W_node_update[:NODE_EMB]
    wu_ea = W_node_update[NODE_EMB:]
    w_all = jnp.concatenate([wa_src, wu_src, wa_tar], axis=1)

    tab = pl.pallas_call(
        _a1_body,
        out_shape=jax.ShapeDtypeStruct((N, 16), f32),
    )(node_feats, node_type_ids.reshape(N, 1), W_node_types, w_all)

    n_blk = E // A2_BLK
    records = pl.pallas_call(
        _a2_body,
        grid=(n_blk,),
        in_specs=[
            pl.BlockSpec((A2_BLK, 16), lambda i: (i, 0)),
            pl.BlockSpec((A2_BLK, 8), lambda i: (i, 0)),
            pl.BlockSpec((16, 32), lambda i: (0, 0)),
            pl.BlockSpec((8, 32), lambda i: (0, 0)),
            pl.BlockSpec((32, 4), lambda i: (0, 0)),
            pl.BlockSpec((32, 4), lambda i: (0, 0)),
            pl.BlockSpec((32, 8), lambda i: (0, 0)),
        ],
        out_specs=pl.BlockSpec((A2_BLK, 16), lambda i: (i, 0)),
        out_shape=jax.ShapeDtypeStruct((E, 16), f32),
    )(edge_attr, edge_type, W_edge_attr, W_edge_type, wa_ea, wa_et, wu_ea)

    mesh = plsc.VectorSubcoreMesh(core_axis_name="c", subcore_axis_name="s")
    dense = pl.kernel(
        _b_body,
        out_type=jax.ShapeDtypeStruct((N * N, 16), f32),
        mesh=mesh,
        compiler_params=pltpu.CompilerParams(use_tc_tiling_on_sc=False,
                                             needs_layout_passes=False),
        scratch_types=[
            pltpu.VMEM((EDGES_PER_TILE, 16), f32),
            pltpu.VMEM((ROWS_PER_TILE, 16), f32),
            pltpu.VMEM((EDGES_PER_TILE,), jnp.int32),
            pltpu.VMEM((EDGES_PER_TILE,), jnp.int32),
            pltpu.VMEM((EDGES_PER_TILE // 128, 128), jnp.int32),
            pltpu.VMEM((N, 16), f32),
            pltpu.VMEM_SHARED((BLK_SLOTS + SC_SUBCORES, 16), f32),
        ],
    )(edge_index, records, tab)

    out = pl.pallas_call(
        _c_body,
        grid=(N // C_BLK,),
        in_specs=[pl.BlockSpec((C_BLK * N // 8, 128), lambda i: (i, 0))],
        out_specs=pl.BlockSpec((C_BLK, HEADS, OUT), lambda i: (i, 0, 0)),
        out_shape=jax.ShapeDtypeStruct((N, HEADS, OUT), f32),
    )(dense.reshape(N * N // 8, 128))
    return out.reshape(N, HEADS * OUT)


# speedup vs baseline: 5.2117x; 1.0108x over previous
import functools

import jax
import jax.numpy as jnp
from jax import lax
from jax.experimental import pallas as pl
from jax.experimental.pallas import tpu as pltpu
from jax.experimental.pallas import tpu_sc as plsc

N = 1024
E = 32768
NODE_EMB = 64
OUT = 8
HEADS = 4
N_NODE_TYPES = 3

A2_BLK = 1024
C_BLK = 64

SC_SUBCORES = 16
SC_CORES = 2
EDGES_PER_TILE = E // SC_SUBCORES
BLK_ROWS = 32
N_BLOCKS = N // BLK_ROWS
BLK_SLOTS = BLK_ROWS * N
ROWS_PER_TILE = BLK_SLOTS // SC_SUBCORES


def _a1_body(nf_ref, tid_ref, wnt_ref, wall_ref, tab_ref):
    nf = nf_ref[...]
    tid = tid_ref[...]
    ne = jnp.zeros((N, NODE_EMB), dtype=jnp.float32)
    for t in range(N_NODE_TYPES):
        emb_t = jnp.dot(nf, wnt_ref[t], preferred_element_type=jnp.float32)
        ne = jnp.where(tid == t, emb_t, ne)
    tab_ref[...] = jnp.dot(ne, wall_ref[...], preferred_element_type=jnp.float32)


def _leaky(x):
    return jnp.where(x >= 0, x, 0.2 * x)


def _a2_body(ea_ref, et_ref, wea_ref, wet_ref,
             waea_ref, waet_ref, wuea_ref, rec_ref):
    ea_emb = _leaky(jnp.dot(ea_ref[...], wea_ref[...],
                            preferred_element_type=jnp.float32))
    et_emb = _leaky(jnp.dot(et_ref[...], wet_ref[...],
                            preferred_element_type=jnp.float32))
    es = (jnp.dot(ea_emb, waea_ref[...], preferred_element_type=jnp.float32)
          + jnp.dot(et_emb, waet_ref[...], preferred_element_type=jnp.float32))
    mu = jnp.dot(ea_emb, wuea_ref[...], preferred_element_type=jnp.float32)
    rec_ref[...] = jnp.concatenate(
        [es, mu, jnp.zeros((A2_BLK, 4), jnp.float32)], axis=1)


def _b_body(ei_ref, rec_hbm, tab_hbm, out_ref,
            rec_v, zeros_v, src_v, dst_v, idx_v, tab_v, shared, sem):
    cid = lax.axis_index("c")
    sid = lax.axis_index("s")
    e0 = sid * EDGES_PER_TILE
    pltpu.sync_copy(rec_hbm.at[pl.ds(e0, EDGES_PER_TILE)], rec_v)
    pltpu.sync_copy(ei_ref.at[0, pl.ds(e0, EDGES_PER_TILE)], src_v)
    pltpu.sync_copy(ei_ref.at[1, pl.ds(e0, EDGES_PER_TILE)], dst_v)
    pltpu.sync_copy(tab_hbm, tab_v)
    junk = BLK_SLOTS + sid

    zk = jnp.zeros((16,), jnp.float32)

    def zero_row(r, carry0):
        zeros_v[r, pl.ds(0, 16)] = zk
        return carry0
    lax.fori_loop(0, ROWS_PER_TILE, zero_row, 0)

    lane = jax.lax.iota(jnp.int32, 16)

    def finish_chunk(c, carry0):
        rows = lane + c * 16
        s16 = src_v[pl.ds(c * 16, 16)]
        d16 = dst_v[pl.ds(c * 16, 16)]
        for ch in range(12):
            chv = jnp.full((16,), ch, jnp.int32)
            v = plsc.load_gather(rec_v, [rows, chv])
            v = v + plsc.load_gather(tab_v, [s16, chv])
            if ch < HEADS:
                v = v + plsc.load_gather(tab_v, [d16, chv + 12])
            v = jnp.where(v >= 0, v, 0.2 * v)
            plsc.store_scatter(rec_v, [rows, chv], v)
        return carry0
    lax.fori_loop(0, EDGES_PER_TILE // 16, finish_chunk, 0)

    def per_block(b, carry):
        gb = cid * (N_BLOCKS // SC_CORES) + b
        pltpu.sync_copy(zeros_v, shared.at[pl.ds(sid * ROWS_PER_TILE, ROWS_PER_TILE)])
        plsc.subcore_barrier()

        def per_chunk(c, carry2):
            s16 = src_v[pl.ds(c * 16, 16)]
            d16 = dst_v[pl.ds(c * 16, 16)]
            inb = (s16 >> 5) == gb
            local = ((s16 & (BLK_ROWS - 1)) << 10) | d16
            iv = jnp.where(inb, local, junk)
            idx_v[c >> 3, pl.ds((c & 7) * 16, 16)] = iv
            return carry2
        lax.fori_loop(0, EDGES_PER_TILE // 16, per_chunk, 0)

        descs = [pltpu.async_copy(rec_v.at[pl.ds(j * 128, 128)],
                                  shared.at[idx_v.at[j]], sem, add=True)
                 for j in range(EDGES_PER_TILE // 128)]
        for d in descs:
            d.wait()
        plsc.subcore_barrier()
        pltpu.sync_copy(
            shared.at[pl.ds(sid * ROWS_PER_TILE, ROWS_PER_TILE)],
            out_ref.at[pl.ds(gb * BLK_SLOTS + sid * ROWS_PER_TILE, ROWS_PER_TILE)])
        plsc.subcore_barrier()
        return carry

    lax.fori_loop(0, N_BLOCKS // SC_CORES, per_block, 0)


def _c_body(d_ref, o_ref):
    x = d_ref[...]
    xt = x.T
    y = xt.reshape(8, 16, C_BLK * N // 8)
    lanes = N // 8

    def plane(c):
        return y[:, c, :].reshape(8, C_BLK, lanes)

    s = jnp.stack([plane(h) for h in range(HEADS)], axis=0)
    sm = jnp.where(s == 0.0, -10000.0, s)
    mx = jnp.max(jnp.max(sm, axis=3), axis=1)
    e = jnp.exp(sm - mx[:, None, :, None])
    z = jnp.sum(jnp.sum(e, axis=3), axis=1)
    m = jnp.stack([plane(HEADS + o) for o in range(OUT)], axis=0)
    ucols = []
    for h in range(HEADS):
        for o in range(OUT):
            t = e[h] * m[o]
            ucols.append(jnp.sum(jnp.sum(t, axis=2), axis=0))
    u = jnp.stack(ucols, axis=1).reshape(C_BLK, HEADS, OUT)
    o_ref[...] = u / z.T[:, :, None]


def kernel(node_feats, edge_index, edge_attr, edge_type, node_type_ids,
           edge_type_ids, W_node_types, W_edge_attr, W_edge_type,
           W_node_update, W_att):
    f32 = jnp.float32
    wa_tar = W_att[0:NODE_EMB]
    wa_ea = W_att[NODE_EMB:NODE_EMB + 32]
    wa_et = W_att[NODE_EMB + 32:NODE_EMB + 64]
    wa_src = W_att[NODE_EMB + 64:]
    wu_src = W_node_update[:NODE_EMB]
    wu_ea = W_node_update[NODE_EMB:]
    w_all = jnp.concatenate([wa_src, wu_src, wa_tar], axis=1)

    tab = pl.pallas_call(
        _a1_body,
        out_shape=jax.ShapeDtypeStruct((N, 16), f32),
    )(node_feats, node_type_ids.reshape(N, 1), W_node_types, w_all)

    n_blk = E // A2_BLK
    records = pl.pallas_call(
        _a2_body,
        grid=(n_blk,),
        in_specs=[
            pl.BlockSpec((A2_BLK, 16), lambda i: (i, 0)),
            pl.BlockSpec((A2_BLK, 8), lambda i: (i, 0)),
            pl.BlockSpec((16, 32), lambda i: (0, 0)),
            pl.BlockSpec((8, 32), lambda i: (0, 0)),
            pl.BlockSpec((32, 4), lambda i: (0, 0)),
            pl.BlockSpec((32, 4), lambda i: (0, 0)),
            pl.BlockSpec((32, 8), lambda i: (0, 0)),
        ],
        out_specs=pl.BlockSpec((A2_BLK, 16), lambda i: (i, 0)),
        out_shape=jax.ShapeDtypeStruct((E, 16), f32),
    )(edge_attr, edge_type, W_edge_attr, W_edge_type, wa_ea, wa_et, wu_ea)

    mesh = plsc.VectorSubcoreMesh(core_axis_name="c", subcore_axis_name="s")
    dense = pl.kernel(
        _b_body,
        out_type=jax.ShapeDtypeStruct((N * N, 16), f32),
        mesh=mesh,
        compiler_params=pltpu.CompilerParams(use_tc_tiling_on_sc=False,
                                             needs_layout_passes=False),
        scratch_types=[
            pltpu.VMEM((EDGES_PER_TILE, 16), f32),
            pltpu.VMEM((ROWS_PER_TILE, 16), f32),
            pltpu.VMEM((EDGES_PER_TILE,), jnp.int32),
            pltpu.VMEM((EDGES_PER_TILE,), jnp.int32),
            pltpu.VMEM((EDGES_PER_TILE // 128, 128), jnp.int32),
            pltpu.VMEM((N, 16), f32),
            pltpu.VMEM_SHARED((BLK_SLOTS + SC_SUBCORES, 16), f32),
            pltpu.SemaphoreType.DMA,
        ],
    )(edge_index, records, tab)

    out = pl.pallas_call(
        _c_body,
        grid=(N // C_BLK,),
        in_specs=[pl.BlockSpec((C_BLK * N // 8, 128), lambda i: (i, 0))],
        out_specs=pl.BlockSpec((C_BLK, HEADS, OUT), lambda i: (i, 0, 0)),
        out_shape=jax.ShapeDtypeStruct((N, HEADS, OUT), f32),
    )(dense.reshape(N * N // 8, 128))
    return out.reshape(N, HEADS * OUT)


# speedup vs baseline: 5.2151x; 1.0007x over previous
import jax
import jax.numpy as jnp
from jax import lax
from jax.experimental import pallas as pl
from jax.experimental.pallas import tpu as pltpu
from jax.experimental.pallas import tpu_sc as plsc

N = 1024
E = 32768
NODE_EMB = 64
OUT = 8
HEADS = 4
N_NODE_TYPES = 3

A2_BLK = 1024
C_BLK = 64

SC_SUBCORES = 16
SC_CORES = 2
EDGES_PER_TILE = E // SC_SUBCORES
BLK_ROWS = 32
N_BLOCKS = N // BLK_ROWS
BLK_SLOTS = BLK_ROWS * N
ROWS_PER_TILE = BLK_SLOTS // SC_SUBCORES


def _a1_body(nf_ref, tid_ref, wnt_ref, wall_ref, tab_ref):
    nf = nf_ref[...]
    tid = tid_ref[...]
    ne = jnp.zeros((N, NODE_EMB), dtype=jnp.float32)
    for t in range(N_NODE_TYPES):
        emb_t = jnp.dot(nf, wnt_ref[t], preferred_element_type=jnp.float32)
        ne = jnp.where(tid == t, emb_t, ne)
    tab_ref[...] = jnp.dot(ne, wall_ref[...], preferred_element_type=jnp.float32)


def _leaky(x):
    return jnp.where(x >= 0, x, 0.2 * x)


def _a2_body(ea_ref, et_ref, wea_ref, wet_ref,
             waea_ref, waet_ref, wuea_ref, rec_ref):
    ea_emb = _leaky(jnp.dot(ea_ref[...], wea_ref[...],
                            preferred_element_type=jnp.float32))
    et_emb = _leaky(jnp.dot(et_ref[...], wet_ref[...],
                            preferred_element_type=jnp.float32))
    es = (jnp.dot(ea_emb, waea_ref[...], preferred_element_type=jnp.float32)
          + jnp.dot(et_emb, waet_ref[...], preferred_element_type=jnp.float32))
    mu = jnp.dot(ea_emb, wuea_ref[...], preferred_element_type=jnp.float32)
    rec_ref[...] = jnp.concatenate(
        [es, mu, jnp.zeros((A2_BLK, 4), jnp.float32)], axis=1)


def _b_body(ei_ref, rec_hbm, tab_hbm, out_ref,
            rec_v, zeros_v, src_v, dst_v, idx_v, tab_v, shared, sem):
    cid = lax.axis_index("c")
    sid = lax.axis_index("s")
    e0 = sid * EDGES_PER_TILE
    pltpu.sync_copy(rec_hbm.at[pl.ds(e0, EDGES_PER_TILE)], rec_v)
    pltpu.sync_copy(ei_ref.at[0, pl.ds(e0, EDGES_PER_TILE)], src_v)
    pltpu.sync_copy(ei_ref.at[1, pl.ds(e0, EDGES_PER_TILE)], dst_v)
    pltpu.sync_copy(tab_hbm, tab_v)
    junk = BLK_SLOTS + sid

    zk = jnp.zeros((16,), jnp.float32)

    def zero_row(r, carry0):
        zeros_v[r, pl.ds(0, 16)] = zk
        return carry0
    lax.fori_loop(0, ROWS_PER_TILE, zero_row, 0)

    lane = jax.lax.iota(jnp.int32, 16)

    def finish_chunk(c, carry0):
        rows = lane + c * 16
        s16 = src_v[pl.ds(c * 16, 16)]
        d16 = dst_v[pl.ds(c * 16, 16)]
        for ch in range(12):
            chv = jnp.full((16,), ch, jnp.int32)
            v = plsc.load_gather(rec_v, [rows, chv])
            v = v + plsc.load_gather(tab_v, [s16, chv])
            if ch < HEADS:
                v = v + plsc.load_gather(tab_v, [d16, chv + 12])
            v = jnp.where(v >= 0, v, 0.2 * v)
            plsc.store_scatter(rec_v, [rows, chv], v)
        return carry0
    lax.fori_loop(0, EDGES_PER_TILE // 16, finish_chunk, 0)

    def per_block(b, carry):
        gb = cid * (N_BLOCKS // SC_CORES) + b
        pltpu.sync_copy(zeros_v, shared.at[pl.ds(sid * ROWS_PER_TILE, ROWS_PER_TILE)])
        plsc.subcore_barrier()

        def per_chunk(c, carry2):
            s16 = src_v[pl.ds(c * 16, 16)]
            d16 = dst_v[pl.ds(c * 16, 16)]
            inb = (s16 >> 5) == gb
            local = ((s16 & (BLK_ROWS - 1)) << 10) | d16
            iv = jnp.where(inb, local, junk)
            idx_v[c >> 3, pl.ds((c & 7) * 16, 16)] = iv
            return carry2
        lax.fori_loop(0, EDGES_PER_TILE // 16, per_chunk, 0)

        descs = [pltpu.async_copy(rec_v.at[pl.ds(j * 128, 128)],
                                  shared.at[idx_v.at[j]], sem, add=True)
                 for j in range(EDGES_PER_TILE // 128)]
        for d in descs:
            d.wait()
        plsc.subcore_barrier()
        pltpu.sync_copy(
            shared.at[pl.ds(sid * ROWS_PER_TILE, ROWS_PER_TILE)],
            out_ref.at[pl.ds(gb * BLK_SLOTS + sid * ROWS_PER_TILE, ROWS_PER_TILE)])
        plsc.subcore_barrier()
        return carry

    lax.fori_loop(0, N_BLOCKS // SC_CORES, per_block, 0)


def _c_body(d_ref, o_ref):
    x = d_ref[...]
    xt = x.T
    y = xt.reshape(8, 16, C_BLK * N // 8)
    lanes = N // 8

    def plane(c):
        return y[:, c, :].reshape(8, C_BLK, lanes)

    s = jnp.stack([plane(h) for h in range(HEADS)], axis=0)
    sm = jnp.where(s == 0.0, -10000.0, s)
    mx = jnp.max(jnp.max(sm, axis=3), axis=1)
    e = jnp.exp(sm - mx[:, None, :, None])
    z = jnp.sum(jnp.sum(e, axis=3), axis=1)
    m = jnp.stack([plane(HEADS + o) for o in range(OUT)], axis=0)
    ucols = []
    for h in range(HEADS):
        for o in range(OUT):
            t = e[h] * m[o]
            ucols.append(jnp.sum(jnp.sum(t, axis=2), axis=0))
    u = jnp.stack(ucols, axis=1).reshape(C_BLK, HEADS, OUT)
    o_ref[...] = u / z.T[:, :, None]


def kernel(node_feats, edge_index, edge_attr, edge_type, node_type_ids,
           edge_type_ids, W_node_types, W_edge_attr, W_edge_type,
           W_node_update, W_att):
    f32 = jnp.float32
    wa_tar = W_att[0:NODE_EMB]
    wa_ea = W_att[NODE_EMB:NODE_EMB + 32]
    wa_et = W_att[NODE_EMB + 32:NODE_EMB + 64]
    wa_src = W_att[NODE_EMB + 64:]
    wu_src = W_node_update[:NODE_EMB]
    wu_ea = W_node_update[NODE_EMB:]
    w_all = jnp.concatenate([wa_src, wu_src, wa_tar], axis=1)

    tab = pl.pallas_call(
        _a1_body,
        out_shape=jax.ShapeDtypeStruct((N, 16), f32),
    )(node_feats, node_type_ids.reshape(N, 1), W_node_types, w_all)

    n_blk = E // A2_BLK
    records = pl.pallas_call(
        _a2_body,
        grid=(n_blk,),
        in_specs=[
            pl.BlockSpec((A2_BLK, 16), lambda i: (i, 0)),
            pl.BlockSpec((A2_BLK, 8), lambda i: (i, 0)),
            pl.BlockSpec((16, 32), lambda i: (0, 0)),
            pl.BlockSpec((8, 32), lambda i: (0, 0)),
            pl.BlockSpec((32, 4), lambda i: (0, 0)),
            pl.BlockSpec((32, 4), lambda i: (0, 0)),
            pl.BlockSpec((32, 8), lambda i: (0, 0)),
        ],
        out_specs=pl.BlockSpec((A2_BLK, 16), lambda i: (i, 0)),
        out_shape=jax.ShapeDtypeStruct((E, 16), f32),
    )(edge_attr, edge_type, W_edge_attr, W_edge_type, wa_ea, wa_et, wu_ea)

    mesh = plsc.VectorSubcoreMesh(core_axis_name="c", subcore_axis_name="s")
    dense = pl.kernel(
        _b_body,
        out_type=jax.ShapeDtypeStruct((N * N, 16), f32),
        mesh=mesh,
        compiler_params=pltpu.CompilerParams(use_tc_tiling_on_sc=False,
                                             needs_layout_passes=False),
        scratch_types=[
            pltpu.VMEM((EDGES_PER_TILE, 16), f32),
            pltpu.VMEM((ROWS_PER_TILE, 16), f32),
            pltpu.VMEM((EDGES_PER_TILE,), jnp.int32),
            pltpu.VMEM((EDGES_PER_TILE,), jnp.int32),
            pltpu.VMEM((EDGES_PER_TILE // 128, 128), jnp.int32),
            pltpu.VMEM((N, 16), f32),
            pltpu.VMEM_SHARED((BLK_SLOTS + SC_SUBCORES, 16), f32),
            pltpu.SemaphoreType.DMA,
        ],
    )(edge_index, records, tab)

    out = pl.pallas_call(
        _c_body,
        grid=(N // C_BLK,),
        in_specs=[pl.BlockSpec((C_BLK * N // 8, 128), lambda i: (i, 0))],
        out_specs=pl.BlockSpec((C_BLK, HEADS, OUT), lambda i: (i, 0, 0)),
        out_shape=jax.ShapeDtypeStruct((N, HEADS, OUT), f32),
    )(dense.reshape(N * N // 8, 128))
    return out.reshape(N, HEADS * OUT)
